# baseline (device time: 160749 ns/iter reference)
import jax
import jax.numpy as jnp
from jax import lax
from jax.experimental import pallas as pl
from jax.experimental.pallas import tpu as pltpu

N_DEV = 4
SCALE = 0.08838834764831843


def kernel(x, Wq, Wo, K_ext, V_ext):
    seq_per = x.shape[1]
    d_model = x.shape[2]
    dh = K_ext.shape[3]
    h_per = Wq.shape[1] // dh

    my = lax.axis_index("i")
    h0 = my * h_per

    xb = x[0].astype(jnp.bfloat16)
    wq = Wq.astype(jnp.bfloat16)
    wo = Wo.astype(jnp.bfloat16)
    k_sl = lax.dynamic_slice_in_dim(K_ext[0], h0, h_per, axis=1)
    v_sl = lax.dynamic_slice_in_dim(V_ext[0], h0, h_per, axis=1)
    kh = jnp.transpose(k_sl, (1, 2, 0)).astype(jnp.bfloat16)
    vh = jnp.transpose(v_sl, (1, 0, 2)).astype(jnp.bfloat16)

    def body(x_ref, wq_ref, wo_ref, k_ref, v_ref, out_ref,
             xg_ref, part_ref, comm_ref, attn_ref,
             ag_send, ag_recv, rs_send, rs_recv):
        my_pos = lax.axis_index("i")
        left = lax.rem(my_pos + N_DEV - 1, N_DEV)
        right = lax.rem(my_pos + 1, N_DEV)

        barrier = pltpu.get_barrier_semaphore()
        for nbr in (left, right):
            pl.semaphore_signal(
                barrier, inc=1,
                device_id=(nbr,), device_id_type=pl.DeviceIdType.MESH,
            )
        pl.semaphore_wait(barrier, 2)

        xg_ref[0] = x_ref[...]
        for h in range(N_DEV - 1):
            rdma = pltpu.make_async_remote_copy(
                src_ref=xg_ref.at[h],
                dst_ref=xg_ref.at[h + 1],
                send_sem=ag_send.at[h],
                recv_sem=ag_recv.at[h],
                device_id=(right,),
                device_id_type=pl.DeviceIdType.MESH,
            )
            rdma.start()
            rdma.wait()

        for j in range(N_DEV):
            xj = xg_ref[j]
            qj = jnp.dot(xj, wq_ref[...], preferred_element_type=jnp.float32)
            qj = (qj * SCALE).astype(jnp.bfloat16)
            for h in range(h_per):
                qh = qj[:, h * dh:(h + 1) * dh]
                s = jnp.dot(qh, k_ref[h],
                            preferred_element_type=jnp.float32)
                m = jnp.max(s, axis=1, keepdims=True)
                p = jnp.exp(s - m)
                l = jnp.sum(p, axis=1, keepdims=True)
                o = jnp.dot(p.astype(jnp.bfloat16), v_ref[h],
                            preferred_element_type=jnp.float32) / l
                attn_ref[:, h * dh:(h + 1) * dh] = o.astype(jnp.bfloat16)
            part_ref[j] = jnp.dot(attn_ref[...], wo_ref[...],
                                  preferred_element_type=jnp.float32)

        for s in range(N_DEV - 1):
            src = part_ref.at[1] if s == 0 else comm_ref.at[(s - 1) % 2]
            rdma = pltpu.make_async_remote_copy(
                src_ref=src,
                dst_ref=comm_ref.at[s % 2],
                send_sem=rs_send.at[s],
                recv_sem=rs_recv.at[s],
                device_id=(right,),
                device_id_type=pl.DeviceIdType.MESH,
            )
            rdma.start()
            rdma.wait()
            if s < N_DEV - 2:
                comm_ref[s % 2] = comm_ref[s % 2] + part_ref[s + 2]
            else:
                out_ref[0] = comm_ref[s % 2] + part_ref[0]

    return pl.pallas_call(
        body,
        out_shape=jax.ShapeDtypeStruct((1, seq_per, d_model), jnp.float32),
        in_specs=[pl.BlockSpec(memory_space=pltpu.VMEM)] * 5,
        out_specs=pl.BlockSpec(memory_space=pltpu.VMEM),
        scratch_shapes=[
            pltpu.VMEM((N_DEV, seq_per, d_model), jnp.bfloat16),
            pltpu.VMEM((N_DEV, seq_per, d_model), jnp.float32),
            pltpu.VMEM((2, seq_per, d_model), jnp.float32),
            pltpu.VMEM((seq_per, d_model), jnp.bfloat16),
            pltpu.SemaphoreType.DMA((N_DEV - 1,)),
            pltpu.SemaphoreType.DMA((N_DEV - 1,)),
            pltpu.SemaphoreType.DMA((N_DEV - 1,)),
            pltpu.SemaphoreType.DMA((N_DEV - 1,)),
        ],
        compiler_params=pltpu.CompilerParams(collective_id=0),
    )(xb, wq, wo, kh, vh)


# device time: 109033 ns/iter; 1.4743x vs baseline; 1.4743x over previous
import jax
import jax.numpy as jnp
from jax import lax
from jax.experimental import pallas as pl
from jax.experimental.pallas import tpu as pltpu

N_DEV = 4
SCALE = 0.08838834764831843


def kernel(x, Wq, Wo, K_ext, V_ext):
    seq_per = x.shape[1]
    d_model = x.shape[2]
    dh = K_ext.shape[3]
    h_per = Wq.shape[1] // dh

    my = lax.axis_index("i")
    h0 = my * h_per

    xb = x[0].astype(jnp.bfloat16)
    wq = Wq.astype(jnp.bfloat16)
    wo = Wo.astype(jnp.bfloat16)
    k_sl = lax.dynamic_slice_in_dim(K_ext[0], h0, h_per, axis=1)
    v_sl = lax.dynamic_slice_in_dim(V_ext[0], h0, h_per, axis=1)
    kh = jnp.transpose(k_sl, (1, 2, 0)).astype(jnp.bfloat16)
    vh = jnp.transpose(v_sl, (1, 0, 2)).astype(jnp.bfloat16)

    def body(x_ref, wq_ref, wo_ref, k_ref, v_ref, out_ref,
             xg_ref, part_ref, comm_ref, attn_ref,
             ag_send, ag_recv, rs_send, rs_recv):
        my_pos = lax.axis_index("i")
        left = lax.rem(my_pos + N_DEV - 1, N_DEV)
        right = lax.rem(my_pos + 1, N_DEV)

        barrier = pltpu.get_barrier_semaphore()
        for nbr in (left, right):
            pl.semaphore_signal(
                barrier, inc=1,
                device_id=(nbr,), device_id_type=pl.DeviceIdType.MESH,
            )
        pl.semaphore_wait(barrier, 2)

        def ag_hop(h):
            return pltpu.make_async_remote_copy(
                src_ref=xg_ref.at[h],
                dst_ref=xg_ref.at[h + 1],
                send_sem=ag_send.at[h],
                recv_sem=ag_recv.at[h],
                device_id=(right,),
                device_id_type=pl.DeviceIdType.MESH,
            )

        def rs_hop(s):
            src = part_ref.at[1] if s == 0 else comm_ref.at[(s - 1) % 2]
            return pltpu.make_async_remote_copy(
                src_ref=src,
                dst_ref=comm_ref.at[s % 2],
                send_sem=rs_send.at[s],
                recv_sem=rs_recv.at[s],
                device_id=(right,),
                device_id_type=pl.DeviceIdType.MESH,
            )

        def compute_chunk(j):
            xj = xg_ref[j]
            qj = jnp.dot(xj, wq_ref[...], preferred_element_type=jnp.float32)
            qj = (qj * SCALE).astype(jnp.bfloat16)
            for h in range(h_per):
                qh = qj[:, h * dh:(h + 1) * dh]
                s = jnp.dot(qh, k_ref[h],
                            preferred_element_type=jnp.float32)
                m = jnp.max(s, axis=1, keepdims=True)
                p = jnp.exp(s - m)
                l = jnp.sum(p, axis=1, keepdims=True)
                o = jnp.dot(p.astype(jnp.bfloat16), v_ref[h],
                            preferred_element_type=jnp.float32) / l
                attn_ref[:, h * dh:(h + 1) * dh] = o.astype(jnp.bfloat16)
            part_ref[j] = jnp.dot(attn_ref[...], wo_ref[...],
                                  preferred_element_type=jnp.float32
                                  ).astype(jnp.bfloat16)

        def rs_accum(slot, j):
            a = comm_ref[slot].astype(jnp.float32)
            b = part_ref[j].astype(jnp.float32)
            comm_ref[slot] = (a + b).astype(jnp.bfloat16)

        xg_ref[0] = x_ref[...]
        ag0 = ag_hop(0)
        ag0.start()
        compute_chunk(0)
        ag0.wait()

        ag1 = ag_hop(1)
        ag1.start()
        compute_chunk(1)
        ag1.wait()

        ag2 = ag_hop(2)
        ag2.start()
        rs0 = rs_hop(0)
        rs0.start()
        compute_chunk(2)
        ag2.wait()
        rs0.wait()

        rs_accum(0, 2)
        rs1 = rs_hop(1)
        rs1.start()
        compute_chunk(3)
        rs1.wait()

        rs_accum(1, 3)
        rs2 = rs_hop(2)
        rs2.start()
        rs2.wait()
        out_ref[0] = (comm_ref[0].astype(jnp.float32)
                      + part_ref[0].astype(jnp.float32))

    return pl.pallas_call(
        body,
        out_shape=jax.ShapeDtypeStruct((1, seq_per, d_model), jnp.float32),
        in_specs=[pl.BlockSpec(memory_space=pltpu.VMEM)] * 5,
        out_specs=pl.BlockSpec(memory_space=pltpu.VMEM),
        scratch_shapes=[
            pltpu.VMEM((N_DEV, seq_per, d_model), jnp.bfloat16),
            pltpu.VMEM((N_DEV, seq_per, d_model), jnp.bfloat16),
            pltpu.VMEM((2, seq_per, d_model), jnp.bfloat16),
            pltpu.VMEM((seq_per, d_model), jnp.bfloat16),
            pltpu.SemaphoreType.DMA((N_DEV - 1,)),
            pltpu.SemaphoreType.DMA((N_DEV - 1,)),
            pltpu.SemaphoreType.DMA((N_DEV - 1,)),
            pltpu.SemaphoreType.DMA((N_DEV - 1,)),
        ],
        compiler_params=pltpu.CompilerParams(collective_id=0),
    )(xb, wq, wo, kh, vh)
